# baseline (speedup 1.0000x reference)
import jax
import jax.numpy as jnp
from jax.experimental import pallas as pl
from jax.experimental.pallas import tpu as pltpu


def _support_kernel(x_ref, w1_ref, o_ref):
    o_ref[...] = jnp.dot(x_ref[...], w1_ref[...],
                         preferred_element_type=jnp.float32)


def _layer1_kernel(adj_ref, sup_ref, b1_ref, w2_ref, o_ref):
    h = jnp.dot(adj_ref[...], sup_ref[...],
                preferred_element_type=jnp.float32)
    h = jnp.maximum(h + b1_ref[...], 0.0)
    o_ref[...] = jnp.dot(h, w2_ref[...],
                         preferred_element_type=jnp.float32)


def _layer2_kernel(adj_ref, s2_ref, b2_ref, o_ref):
    logits = jnp.dot(adj_ref[...], s2_ref[...],
                     preferred_element_type=jnp.float32) + b2_ref[...]
    m = jnp.max(logits, axis=1, keepdims=True)
    z = logits - m
    lse = jnp.log(jnp.sum(jnp.exp(z), axis=1, keepdims=True))
    o_ref[...] = z - lse


def kernel(x, adj, W1, b1, W2, b2):
    n, _ = x.shape
    h_dim = W1.shape[1]
    c_dim = W2.shape[1]
    bm = 200

    b1_2d = b1.reshape(1, h_dim)
    b2_2d = b2.reshape(1, c_dim)

    support = pl.pallas_call(
        _support_kernel,
        out_shape=jax.ShapeDtypeStruct((n, h_dim), jnp.float32),
    )(x, W1)

    grid = (n // bm,)
    params = pltpu.CompilerParams(dimension_semantics=("parallel",))

    s2 = pl.pallas_call(
        _layer1_kernel,
        grid=grid,
        in_specs=[
            pl.BlockSpec((bm, n), lambda i: (i, 0)),
            pl.BlockSpec((n, h_dim), lambda i: (0, 0)),
            pl.BlockSpec((1, h_dim), lambda i: (0, 0)),
            pl.BlockSpec((h_dim, c_dim), lambda i: (0, 0)),
        ],
        out_specs=pl.BlockSpec((bm, c_dim), lambda i: (i, 0)),
        out_shape=jax.ShapeDtypeStruct((n, c_dim), jnp.float32),
        compiler_params=params,
    )(adj, support, b1_2d, W2)

    out = pl.pallas_call(
        _layer2_kernel,
        grid=grid,
        in_specs=[
            pl.BlockSpec((bm, n), lambda i: (i, 0)),
            pl.BlockSpec((n, c_dim), lambda i: (0, 0)),
            pl.BlockSpec((1, c_dim), lambda i: (0, 0)),
        ],
        out_specs=pl.BlockSpec((bm, c_dim), lambda i: (i, 0)),
        out_shape=jax.ShapeDtypeStruct((n, c_dim), jnp.float32),
        compiler_params=params,
    )(adj, s2, b2_2d)

    return out


# speedup vs baseline: 1.0195x; 1.0195x over previous
import jax
import jax.numpy as jnp
from jax.experimental import pallas as pl
from jax.experimental.pallas import tpu as pltpu


def _support_kernel(x_ref, w1_ref, o_ref):
    o_ref[...] = jnp.dot(x_ref[...], w1_ref[...],
                         preferred_element_type=jnp.float32)


def _layer1_kernel(adj_ref, sup_ref, b1_ref, w2_ref, o_ref):
    h = jnp.dot(adj_ref[...], sup_ref[...],
                preferred_element_type=jnp.float32)
    h = jnp.maximum(h + b1_ref[...], 0.0)
    o_ref[...] = jnp.dot(h, w2_ref[...],
                         preferred_element_type=jnp.float32)


def _layer2_kernel(adj_ref, s2_ref, b2_ref, o_ref):
    logits = jnp.dot(adj_ref[...], s2_ref[...],
                     preferred_element_type=jnp.float32) + b2_ref[...]
    m = jnp.max(logits, axis=1, keepdims=True)
    z = logits - m
    lse = jnp.log(jnp.sum(jnp.exp(z), axis=1, keepdims=True))
    o_ref[...] = z - lse


def kernel(x, adj, W1, b1, W2, b2):
    n, _ = x.shape
    h_dim = W1.shape[1]
    c_dim = W2.shape[1]
    bm = 400

    b1_2d = b1.reshape(1, h_dim)
    b2_2d = b2.reshape(1, c_dim)

    support = pl.pallas_call(
        _support_kernel,
        out_shape=jax.ShapeDtypeStruct((n, h_dim), jnp.float32),
    )(x, W1)

    grid = (n // bm,)
    params = pltpu.CompilerParams(dimension_semantics=("parallel",))

    s2 = pl.pallas_call(
        _layer1_kernel,
        grid=grid,
        in_specs=[
            pl.BlockSpec((bm, n), lambda i: (i, 0)),
            pl.BlockSpec((n, h_dim), lambda i: (0, 0)),
            pl.BlockSpec((1, h_dim), lambda i: (0, 0)),
            pl.BlockSpec((h_dim, c_dim), lambda i: (0, 0)),
        ],
        out_specs=pl.BlockSpec((bm, c_dim), lambda i: (i, 0)),
        out_shape=jax.ShapeDtypeStruct((n, c_dim), jnp.float32),
        compiler_params=params,
    )(adj, support, b1_2d, W2)

    out = pl.pallas_call(
        _layer2_kernel,
        grid=grid,
        in_specs=[
            pl.BlockSpec((bm, n), lambda i: (i, 0)),
            pl.BlockSpec((n, c_dim), lambda i: (0, 0)),
            pl.BlockSpec((1, c_dim), lambda i: (0, 0)),
        ],
        out_specs=pl.BlockSpec((bm, c_dim), lambda i: (i, 0)),
        out_shape=jax.ShapeDtypeStruct((n, c_dim), jnp.float32),
        compiler_params=params,
    )(adj, s2, b2_2d)

    return out
